# Initial kernel scaffold; baseline (speedup 1.0000x reference)
import jax
import jax.numpy as jnp
from jax.experimental import pallas as pl


def kernel(idxTensor, boxes, scores):
    raise NotImplementedError("write your pallas kernel here")


# speedup vs baseline: 1.8904x; 1.8904x over previous
import functools

import jax
import jax.numpy as jnp
from jax import lax
from jax.experimental import pallas as pl
from jax.experimental.pallas import tpu as pltpu
from jax.experimental.pallas import tpu_sc as plsc

A = 33600
N = 20000
C = 80

NC = 2
NS = 16
NW = NC * NS
BPW = 640
B_PAD = BPW * NW
D = 6
CHUNK = 128
NCHUNK = BPW * D // CHUNK

BLK = 1024


def _tc_reduce_body(s_ref, maxv_ref, argc_ref):
    s = s_ref[...]
    m = jnp.max(s, axis=0, keepdims=True)
    iot = lax.broadcasted_iota(jnp.int32, s.shape, 0)
    a = jnp.min(jnp.where(s == m, iot, C), axis=0, keepdims=True)
    maxv_ref[...] = m
    argc_ref[...] = a.astype(jnp.float32)


_tc_reduce = pl.pallas_call(
    _tc_reduce_body,
    grid=(pl.cdiv(A, BLK),),
    in_specs=[pl.BlockSpec((C, BLK), lambda i: (0, i))],
    out_specs=[
        pl.BlockSpec((1, BLK), lambda i: (0, i)),
        pl.BlockSpec((1, BLK), lambda i: (0, i)),
    ],
    out_shape=[
        jax.ShapeDtypeStruct((1, A), jnp.float32),
        jax.ShapeDtypeStruct((1, A), jnp.float32),
    ],
)


def _sc_gather_body(idx_hbm, tbl_hbm, out_hbm, idx_v, rows_v, sem):
    wid = lax.axis_index("s") * NC + lax.axis_index("c")
    pltpu.sync_copy(idx_hbm.at[wid], idx_v)
    copies = [
        pltpu.async_copy(tbl_hbm.at[idx_v.at[k]], rows_v.at[k], sem)
        for k in range(NCHUNK)
    ]
    for cp in copies:
        cp.wait()
    pltpu.sync_copy(rows_v, out_hbm.at[wid])


@functools.cache
def _make_sc_gather():
    mesh = plsc.VectorSubcoreMesh(
        core_axis_name="c", subcore_axis_name="s",
        num_cores=NC, num_subcores=NS,
    )
    return pl.kernel(
        _sc_gather_body,
        out_type=jax.ShapeDtypeStruct((NW, NCHUNK, CHUNK), jnp.float32),
        mesh=mesh,
        scratch_types=[
            pltpu.VMEM((NCHUNK, CHUNK), jnp.int32),
            pltpu.VMEM((NCHUNK, CHUNK), jnp.float32),
            pltpu.SemaphoreType.DMA,
        ],
    )


def kernel(idxTensor, boxes, scores):
    maxv, argc = _tc_reduce(scores[0])
    tbl = jnp.concatenate(
        [boxes.reshape(4 * A), maxv.reshape(A), argc.reshape(A)]
    )
    idx = idxTensor[:, 2]
    idx_pad = jnp.concatenate([idx, jnp.zeros((B_PAD - N,), jnp.int32)])
    idx6 = (idx_pad[:, None] + (jnp.arange(D, dtype=jnp.int32) * A)[None, :])
    rows = _make_sc_gather()(
        idx6.reshape(NW, NCHUNK, CHUNK), tbl
    )
    rows = rows.reshape(B_PAD, D)[:N]
    bbox_result = rows[:, :4][None]
    score_result = rows[:, 4][None]
    classes_result = rows[:, 5].astype(jnp.int32)[None]
    num_dets = jnp.array(N, jnp.int32)
    return (bbox_result, score_result, classes_result, num_dets)


# speedup vs baseline: 3.9266x; 2.0771x over previous
import functools

import jax
import jax.numpy as jnp
from jax import lax
from jax.experimental import pallas as pl
from jax.experimental.pallas import tpu as pltpu
from jax.experimental.pallas import tpu_sc as plsc

A = 33600
N = 20000
C = 80

NC = 2
NS = 16
NW = NC * NS
BPW = 640
B_PAD = BPW * NW
LANES = 16
CHUNK = 128
KPW = BPW // CHUNK

BLK = 1024


def _tc_reduce_body(s_ref, maxv_ref, argc_ref):
    s = s_ref[...]
    m = jnp.max(s, axis=0, keepdims=True)
    iot = lax.broadcasted_iota(jnp.int32, s.shape, 0)
    a = jnp.min(jnp.where(s == m, iot, C), axis=0, keepdims=True)
    maxv_ref[...] = m
    argc_ref[...] = a.astype(jnp.float32)


_tc_reduce = pl.pallas_call(
    _tc_reduce_body,
    grid=(pl.cdiv(A, BLK),),
    in_specs=[pl.BlockSpec((C, BLK), lambda i: (0, i))],
    out_specs=[
        pl.BlockSpec((1, BLK), lambda i: (0, i)),
        pl.BlockSpec((1, BLK), lambda i: (0, i)),
    ],
    out_shape=[
        jax.ShapeDtypeStruct((1, A), jnp.float32),
        jax.ShapeDtypeStruct((1, A), jnp.float32),
    ],
)


def _sc_gather_body(idx_hbm, boxes_hbm, maxv_hbm, argc_hbm, out_hbm,
                    idx_v, ib1, ib2, ib3, rows_v, sem):
    wid = lax.axis_index("s") * NC + lax.axis_index("c")
    base = wid * BPW
    pltpu.sync_copy(idx_hbm.at[wid], idx_v)
    for j, ib in ((1, ib1), (2, ib2), (3, ib3)):
        off = jnp.full((LANES,), j * A, jnp.int32)
        for k in range(KPW):
            for c in range(CHUNK // LANES):
                sl = pl.ds(c * LANES, LANES)
                ib[k, sl] = idx_v[k, sl] + off
    fields = (
        (boxes_hbm, idx_v), (boxes_hbm, ib1), (boxes_hbm, ib2),
        (boxes_hbm, ib3), (maxv_hbm, idx_v), (argc_hbm, idx_v),
    )
    copies = [
        pltpu.async_copy(
            tbl.at[ib.at[k]],
            rows_v.at[pl.ds((f * KPW + k) * CHUNK, CHUNK)],
            sem,
        )
        for f, (tbl, ib) in enumerate(fields)
        for k in range(KPW)
    ]
    for cp in copies:
        cp.wait()
    for f in range(6):
        pltpu.sync_copy(
            rows_v.at[pl.ds(f * BPW, BPW)],
            out_hbm.at[f, pl.ds(base, BPW)],
        )


@functools.cache
def _make_sc_gather():
    mesh = plsc.VectorSubcoreMesh(
        core_axis_name="c", subcore_axis_name="s",
        num_cores=NC, num_subcores=NS,
    )
    return pl.kernel(
        _sc_gather_body,
        out_type=jax.ShapeDtypeStruct((6, B_PAD), jnp.float32),
        mesh=mesh,
        scratch_types=[
            pltpu.VMEM((KPW, CHUNK), jnp.int32),
            pltpu.VMEM((KPW, CHUNK), jnp.int32),
            pltpu.VMEM((KPW, CHUNK), jnp.int32),
            pltpu.VMEM((KPW, CHUNK), jnp.int32),
            pltpu.VMEM((6 * BPW,), jnp.float32),
            pltpu.SemaphoreType.DMA,
        ],
    )


def kernel(idxTensor, boxes, scores):
    maxv, argc = _tc_reduce(scores[0])
    idx = idxTensor[:, 2]
    idx_pad = jnp.concatenate([idx, jnp.zeros((B_PAD - N,), jnp.int32)])
    rows = _make_sc_gather()(
        idx_pad.reshape(NW, KPW, CHUNK),
        boxes.reshape(4 * A),
        maxv.reshape(A),
        argc.reshape(A),
    )
    bbox_result = rows[:4, :N].T[None]
    score_result = rows[4, :N][None]
    classes_result = rows[5, :N].astype(jnp.int32)[None]
    num_dets = jnp.array(N, jnp.int32)
    return (bbox_result, score_result, classes_result, num_dets)


# speedup vs baseline: 4.8416x; 1.2330x over previous
import functools

import jax
import jax.numpy as jnp
from jax import lax
from jax.experimental import pallas as pl
from jax.experimental.pallas import tpu as pltpu
from jax.experimental.pallas import tpu_sc as plsc

A = 33600
N = 20000
C = 80

NC = 2
NS = 16
NW = NC * NS
BPW = 640
B_PAD = BPW * NW
LANES = 16
CHUNK = 128
KPW = BPW // CHUNK

BLK = 4224


def _tc_reduce_body(s_ref, maxv_ref, argc_ref):
    s = s_ref[...]
    m = jnp.max(s, axis=0, keepdims=True)
    iot = lax.broadcasted_iota(jnp.int32, s.shape, 0)
    a = jnp.min(jnp.where(s == m, iot, C), axis=0, keepdims=True)
    maxv_ref[...] = m
    argc_ref[...] = a.astype(jnp.float32)


_tc_reduce = pl.pallas_call(
    _tc_reduce_body,
    grid=(pl.cdiv(A, BLK),),
    in_specs=[pl.BlockSpec((C, BLK), lambda i: (0, i))],
    out_specs=[
        pl.BlockSpec((1, BLK), lambda i: (0, i)),
        pl.BlockSpec((1, BLK), lambda i: (0, i)),
    ],
    out_shape=[
        jax.ShapeDtypeStruct((1, A), jnp.float32),
        jax.ShapeDtypeStruct((1, A), jnp.float32),
    ],
)


def _sc_gather_body(idx_hbm, boxes_hbm, maxv_hbm, argc_hbm, out_hbm,
                    idx_v, ib1, ib2, ib3, rows_v, sem):
    wid = lax.axis_index("s") * NC + lax.axis_index("c")
    base = wid * BPW
    pltpu.sync_copy(idx_hbm.at[wid], idx_v)
    for j, ib in ((1, ib1), (2, ib2), (3, ib3)):
        off = jnp.full((LANES,), j * A, jnp.int32)
        for k in range(KPW):
            for c in range(CHUNK // LANES):
                sl = pl.ds(c * LANES, LANES)
                ib[k, sl] = idx_v[k, sl] + off
    fields = (
        (boxes_hbm, idx_v), (boxes_hbm, ib1), (boxes_hbm, ib2),
        (boxes_hbm, ib3), (maxv_hbm, idx_v), (argc_hbm, idx_v),
    )
    copies = [
        pltpu.async_copy(
            tbl.at[ib.at[k]],
            rows_v.at[pl.ds((f * KPW + k) * CHUNK, CHUNK)],
            sem,
        )
        for f, (tbl, ib) in enumerate(fields)
        for k in range(KPW)
    ]
    for cp in copies:
        cp.wait()
    for f in range(6):
        pltpu.sync_copy(
            rows_v.at[pl.ds(f * BPW, BPW)],
            out_hbm.at[f, pl.ds(base, BPW)],
        )


@functools.cache
def _make_sc_gather():
    mesh = plsc.VectorSubcoreMesh(
        core_axis_name="c", subcore_axis_name="s",
        num_cores=NC, num_subcores=NS,
    )
    return pl.kernel(
        _sc_gather_body,
        out_type=jax.ShapeDtypeStruct((6, B_PAD), jnp.float32),
        mesh=mesh,
        scratch_types=[
            pltpu.VMEM((KPW, CHUNK), jnp.int32),
            pltpu.VMEM((KPW, CHUNK), jnp.int32),
            pltpu.VMEM((KPW, CHUNK), jnp.int32),
            pltpu.VMEM((KPW, CHUNK), jnp.int32),
            pltpu.VMEM((6 * BPW,), jnp.float32),
            pltpu.SemaphoreType.DMA,
        ],
    )


def kernel(idxTensor, boxes, scores):
    maxv, argc = _tc_reduce(scores[0])
    idx = idxTensor[:, 2]
    idx_pad = jnp.concatenate([idx, jnp.zeros((B_PAD - N,), jnp.int32)])
    rows = _make_sc_gather()(
        idx_pad.reshape(NW, KPW, CHUNK),
        boxes.reshape(4 * A),
        maxv.reshape(A),
        argc.reshape(A),
    )
    bbox_result = rows[:4, :N].T[None]
    score_result = rows[4, :N][None]
    classes_result = rows[5, :N].astype(jnp.int32)[None]
    num_dets = jnp.array(N, jnp.int32)
    return (bbox_result, score_result, classes_result, num_dets)


# speedup vs baseline: 5.0219x; 1.0372x over previous
import functools

import jax
import jax.numpy as jnp
from jax import lax
from jax.experimental import pallas as pl
from jax.experimental.pallas import tpu as pltpu
from jax.experimental.pallas import tpu_sc as plsc

A = 33600
N = 20000
C = 80

NC = 2
NS = 16
NW = NC * NS
BPW = 640
B_PAD = BPW * NW
LANES = 16
CHUNK = 128
KPW = BPW // CHUNK

BLK = 8448


def _tc_reduce_body(s_ref, maxv_ref, argc_ref):
    s = s_ref[...]
    m = jnp.max(s, axis=0, keepdims=True)
    iot = lax.broadcasted_iota(jnp.int32, s.shape, 0)
    a = jnp.min(jnp.where(s == m, iot, C), axis=0, keepdims=True)
    maxv_ref[...] = m
    argc_ref[...] = a.astype(jnp.float32)


_tc_reduce = pl.pallas_call(
    _tc_reduce_body,
    grid=(pl.cdiv(A, BLK),),
    in_specs=[pl.BlockSpec((C, BLK), lambda i: (0, i))],
    out_specs=[
        pl.BlockSpec((1, BLK), lambda i: (0, i)),
        pl.BlockSpec((1, BLK), lambda i: (0, i)),
    ],
    out_shape=[
        jax.ShapeDtypeStruct((1, A), jnp.float32),
        jax.ShapeDtypeStruct((1, A), jnp.float32),
    ],
)


def _sc_gather_body(idx_hbm, boxes_hbm, maxv_hbm, argc_hbm, out_hbm,
                    idx_v, ib1, ib2, ib3, rows_v, sem):
    wid = lax.axis_index("s") * NC + lax.axis_index("c")
    base = wid * BPW
    pltpu.sync_copy(idx_hbm.at[wid], idx_v)
    for j, ib in ((1, ib1), (2, ib2), (3, ib3)):
        off = jnp.full((LANES,), j * A, jnp.int32)
        for k in range(KPW):
            for c in range(CHUNK // LANES):
                sl = pl.ds(c * LANES, LANES)
                ib[k, sl] = idx_v[k, sl] + off
    fields = (
        (boxes_hbm, idx_v), (boxes_hbm, ib1), (boxes_hbm, ib2),
        (boxes_hbm, ib3), (maxv_hbm, idx_v), (argc_hbm, idx_v),
    )
    copies = [
        pltpu.async_copy(
            tbl.at[ib.at[k]],
            rows_v.at[pl.ds((f * KPW + k) * CHUNK, CHUNK)],
            sem,
        )
        for f, (tbl, ib) in enumerate(fields)
        for k in range(KPW)
    ]
    for cp in copies:
        cp.wait()
    for f in range(6):
        pltpu.sync_copy(
            rows_v.at[pl.ds(f * BPW, BPW)],
            out_hbm.at[f, pl.ds(base, BPW)],
        )


@functools.cache
def _make_sc_gather():
    mesh = plsc.VectorSubcoreMesh(
        core_axis_name="c", subcore_axis_name="s",
        num_cores=NC, num_subcores=NS,
    )
    return pl.kernel(
        _sc_gather_body,
        out_type=jax.ShapeDtypeStruct((6, B_PAD), jnp.float32),
        mesh=mesh,
        scratch_types=[
            pltpu.VMEM((KPW, CHUNK), jnp.int32),
            pltpu.VMEM((KPW, CHUNK), jnp.int32),
            pltpu.VMEM((KPW, CHUNK), jnp.int32),
            pltpu.VMEM((KPW, CHUNK), jnp.int32),
            pltpu.VMEM((6 * BPW,), jnp.float32),
            pltpu.SemaphoreType.DMA,
        ],
    )


def kernel(idxTensor, boxes, scores):
    maxv, argc = _tc_reduce(scores[0])
    idx = idxTensor[:, 2]
    idx_pad = jnp.concatenate([idx, jnp.zeros((B_PAD - N,), jnp.int32)])
    rows = _make_sc_gather()(
        idx_pad.reshape(NW, KPW, CHUNK),
        boxes.reshape(4 * A),
        maxv.reshape(A),
        argc.reshape(A),
    )
    bbox_result = rows[:4, :N].T[None]
    score_result = rows[4, :N][None]
    classes_result = rows[5, :N].astype(jnp.int32)[None]
    num_dets = jnp.array(N, jnp.int32)
    return (bbox_result, score_result, classes_result, num_dets)


# speedup vs baseline: 5.3760x; 1.0705x over previous
import functools

import jax
import jax.numpy as jnp
from jax import lax
from jax.experimental import pallas as pl
from jax.experimental.pallas import tpu as pltpu
from jax.experimental.pallas import tpu_sc as plsc

A = 33600
N = 20000
C = 80

NC = 2
NS = 16
NW = NC * NS
BPW = 640
B_PAD = BPW * NW
LANES = 16
CHUNK = 128
KPW = BPW // CHUNK

BLK = 8448


def _tc_reduce_body(s_ref, maxv_ref, argc_ref):
    s = s_ref[...]
    m = jnp.max(s, axis=0, keepdims=True)
    iot = lax.broadcasted_iota(jnp.int32, s.shape, 0)
    a = jnp.min(jnp.where(s == m, iot, C), axis=0, keepdims=True)
    maxv_ref[...] = m
    argc_ref[...] = a.astype(jnp.float32)


_tc_reduce = pl.pallas_call(
    _tc_reduce_body,
    grid=(pl.cdiv(A, BLK),),
    in_specs=[pl.BlockSpec((C, BLK), lambda i: (0, i))],
    out_specs=[
        pl.BlockSpec((1, BLK), lambda i: (0, i)),
        pl.BlockSpec((1, BLK), lambda i: (0, i)),
    ],
    out_shape=[
        jax.ShapeDtypeStruct((1, A), jnp.float32),
        jax.ShapeDtypeStruct((1, A), jnp.float32),
    ],
)


def _sc_boxes_body(idx_hbm, boxes_hbm, out_hbm,
                   idx_v, ib1, ib2, ib3, rows_v, sem):
    wid = lax.axis_index("s") * NC + lax.axis_index("c")
    base = wid * BPW
    pltpu.sync_copy(idx_hbm.at[wid], idx_v)
    for j, ib in ((1, ib1), (2, ib2), (3, ib3)):
        off = jnp.full((LANES,), j * A, jnp.int32)
        for k in range(KPW):
            for c in range(CHUNK // LANES):
                sl = pl.ds(c * LANES, LANES)
                ib[k, sl] = idx_v[k, sl] + off
    fields = ((boxes_hbm, idx_v), (boxes_hbm, ib1),
              (boxes_hbm, ib2), (boxes_hbm, ib3))
    copies = [
        pltpu.async_copy(
            tbl.at[ib.at[k]],
            rows_v.at[pl.ds((f * KPW + k) * CHUNK, CHUNK)],
            sem,
        )
        for f, (tbl, ib) in enumerate(fields)
        for k in range(KPW)
    ]
    for cp in copies:
        cp.wait()
    for f in range(4):
        pltpu.sync_copy(
            rows_v.at[pl.ds(f * BPW, BPW)],
            out_hbm.at[f, pl.ds(base, BPW)],
        )


def _sc_mvac_body(idx_hbm, maxv_hbm, argc_hbm, out_hbm,
                  idx_v, rows_v, sem):
    wid = lax.axis_index("s") * NC + lax.axis_index("c")
    base = wid * BPW
    pltpu.sync_copy(idx_hbm.at[wid], idx_v)
    copies = [
        pltpu.async_copy(
            tbl.at[idx_v.at[k]],
            rows_v.at[pl.ds((f * KPW + k) * CHUNK, CHUNK)],
            sem,
        )
        for f, tbl in enumerate((maxv_hbm, argc_hbm))
        for k in range(KPW)
    ]
    for cp in copies:
        cp.wait()
    for f in range(2):
        pltpu.sync_copy(
            rows_v.at[pl.ds(f * BPW, BPW)],
            out_hbm.at[f, pl.ds(base, BPW)],
        )


@functools.cache
def _make_sc_kernels():
    mesh = plsc.VectorSubcoreMesh(
        core_axis_name="c", subcore_axis_name="s",
        num_cores=NC, num_subcores=NS,
    )
    boxes_k = pl.kernel(
        _sc_boxes_body,
        out_type=jax.ShapeDtypeStruct((4, B_PAD), jnp.float32),
        mesh=mesh,
        scratch_types=[
            pltpu.VMEM((KPW, CHUNK), jnp.int32),
            pltpu.VMEM((KPW, CHUNK), jnp.int32),
            pltpu.VMEM((KPW, CHUNK), jnp.int32),
            pltpu.VMEM((KPW, CHUNK), jnp.int32),
            pltpu.VMEM((4 * BPW,), jnp.float32),
            pltpu.SemaphoreType.DMA,
        ],
    )
    mvac_k = pl.kernel(
        _sc_mvac_body,
        out_type=jax.ShapeDtypeStruct((2, B_PAD), jnp.float32),
        mesh=mesh,
        scratch_types=[
            pltpu.VMEM((KPW, CHUNK), jnp.int32),
            pltpu.VMEM((2 * BPW,), jnp.float32),
            pltpu.SemaphoreType.DMA,
        ],
    )
    return boxes_k, mvac_k


def kernel(idxTensor, boxes, scores):
    boxes_k, mvac_k = _make_sc_kernels()
    idx = idxTensor[:, 2]
    idx_pad = jnp.concatenate([idx, jnp.zeros((B_PAD - N,), jnp.int32)])
    idx3 = idx_pad.reshape(NW, KPW, CHUNK)
    rows_b = boxes_k(idx3, boxes.reshape(4 * A))
    maxv, argc = _tc_reduce(scores[0])
    rows_s = mvac_k(idx3, maxv.reshape(A), argc.reshape(A))
    bbox_result = rows_b[:, :N].T[None]
    score_result = rows_s[0, :N][None]
    classes_result = rows_s[1, :N].astype(jnp.int32)[None]
    num_dets = jnp.array(N, jnp.int32)
    return (bbox_result, score_result, classes_result, num_dets)
